# baseline (speedup 1.0000x reference)
import functools

import jax
import jax.numpy as jnp
from jax import lax
from jax.experimental import pallas as pl
from jax.experimental.pallas import tpu as pltpu
from jax.experimental.pallas import tpu_sc as plsc

B = 4096
L = 200
EMB = 64
H = 300

NC, NS, LANES = 2, 16, 16
NW = NC * NS
SEG_PER_W = B // NW
CH0, CH1 = 128, L - 128
ECH = EMB // LANES

_mesh = plsc.VectorSubcoreMesh(core_axis_name="c", subcore_axis_name="s")


@functools.partial(
    pl.kernel,
    out_type=jax.ShapeDtypeStruct((B, EMB), jnp.float32),
    mesh=_mesh,
    scratch_types=[
        pltpu.VMEM((SEG_PER_W, L), jnp.int32),
        pltpu.VMEM((L, EMB), jnp.float32),
        pltpu.VMEM((L, EMB), jnp.float32),
        pltpu.VMEM((L, EMB), jnp.float32),
        pltpu.VMEM((L, EMB), jnp.float32),
        pltpu.VMEM((L, EMB), jnp.float32),
        pltpu.VMEM((L, EMB), jnp.float32),
        pltpu.VMEM((SEG_PER_W, EMB), jnp.float32),
        pltpu.SemaphoreType.DMA,
        pltpu.SemaphoreType.DMA,
        pltpu.SemaphoreType.DMA,
        pltpu.SemaphoreType.DMA,
        pltpu.SemaphoreType.DMA,
        pltpu.SemaphoreType.DMA,
    ],
    compiler_params=pltpu.CompilerParams(use_tc_tiling_on_sc=False),
)
def _sc_pool(ids_hbm, table_hbm, out_hbm, idx_v, buf0, buf1, buf2, buf3,
             buf4, buf5, out_v, sem0, sem1, sem2, sem3, sem4, sem5):
    wid = lax.axis_index("s") * NC + lax.axis_index("c")
    pltpu.sync_copy(ids_hbm.at[pl.ds(wid * SEG_PER_W, SEG_PER_W)], idx_v)

    bufs = (buf0, buf1, buf2, buf3, buf4, buf5)
    sems = (sem0, sem1, sem2, sem3, sem4, sem5)
    NBUF = 6

    def fire(seg, b):
        pltpu.async_copy(table_hbm.at[idx_v.at[seg, pl.ds(0, CH0)]],
                         bufs[b].at[pl.ds(0, CH0)], sems[b])
        pltpu.async_copy(table_hbm.at[idx_v.at[seg, pl.ds(CH0, CH1)]],
                         bufs[b].at[pl.ds(CH0, CH1)], sems[b])

    def drain(b):
        pltpu.make_async_copy(table_hbm.at[pl.ds(0, L)], bufs[b], sems[b]).wait()

    for b in range(NBUF):
        fire(b, b)

    def seg_body(s, b):
        drain(b)
        buf = bufs[b]

        def row8(r8, acc):
            for rr in range(8):
                r = r8 * 8 + rr
                acc = tuple(acc[c] + buf[r, pl.ds(c * LANES, LANES)]
                            for c in range(ECH))
            return acc

        acc = tuple(jnp.zeros((LANES,), jnp.float32) for _ in range(ECH))
        acc = lax.fori_loop(0, L // 8, row8, acc)
        for c in range(ECH):
            out_v[s, pl.ds(c * LANES, LANES)] = acc[c]

        nxt = s + NBUF

        @pl.when(nxt < SEG_PER_W)
        def _():
            fire(nxt, b)

    def loop_body(i, carry):
        for b in range(NBUF):
            seg_body(NBUF * i + b, b)
        return carry

    lax.fori_loop(0, SEG_PER_W // NBUF, loop_body, 0)
    for b in range(SEG_PER_W % NBUF):
        seg_body((SEG_PER_W // NBUF) * NBUF + b, b)
    pltpu.sync_copy(out_v, out_hbm.at[pl.ds(wid * SEG_PER_W, SEG_PER_W)])


VOCAB = 1000000
TCOLS = 32768
THALF = TCOLS // 2
LOG_THALF = THALF.bit_length() - 1
TGRID = (VOCAB + TCOLS - 1) // TCOLS
VOCAB_PAD = TGRID * TCOLS


def _tr_body(xt_ref, out_ref):
    x = xt_ref[...]
    xx = jnp.concatenate([x[:, :THALF], x[:, THALF:]], axis=0)
    out_ref[...] = jnp.transpose(xx)


_tr_call = pl.pallas_call(
    _tr_body,
    grid=(TGRID,),
    in_specs=[pl.BlockSpec((EMB, TCOLS), lambda i: (0, i))],
    out_specs=pl.BlockSpec((THALF, 2 * EMB), lambda i: (i, 0)),
    out_shape=jax.ShapeDtypeStruct((TGRID * THALF, 2 * EMB), jnp.float32),
)


BLK = 512
GRID = B // BLK


def _mlp_body(x_ref, len_ref, lab_ref, w1_ref, b1_ref, w2_ref, b2_ref,
              w3_ref, b3_ref, wfc_ref, bfc_ref, loss_ref, pred_ref):
    i = pl.program_id(0)
    x = x_ref[...] * (1.0 / len_ref[...])
    h = jnp.tanh(jnp.dot(x, w1_ref[...], preferred_element_type=jnp.float32)
                 + b1_ref[...])
    h = jnp.tanh(jnp.dot(h, w2_ref[...], preferred_element_type=jnp.float32)
                 + b2_ref[...])
    h = jnp.tanh(jnp.dot(h, w3_ref[...], preferred_element_type=jnp.float32)
                 + b3_ref[...])
    z = jnp.dot(h, wfc_ref[...], preferred_element_type=jnp.float32) + bfc_ref[...]
    pred_ref[...] = (z > 0.0).astype(jnp.float32)
    pv = jnp.clip(jax.nn.sigmoid(z), 1e-7, 1.0 - 1e-7)
    lv = lab_ref[...]
    part = -jnp.sum(lv * jnp.log(pv) + (1.0 - lv) * jnp.log(1.0 - pv)) * (1.0 / B)

    @pl.when(i == 0)
    def _():
        loss_ref[...] = jnp.zeros((1, 1), jnp.float32)

    loss_ref[...] += jnp.reshape(part, (1, 1))


_mlp_call = pl.pallas_call(
    _mlp_body,
    grid=(GRID,),
    in_specs=[
        pl.BlockSpec((BLK, EMB), lambda i: (i, 0)),
        pl.BlockSpec((BLK, 1), lambda i: (i, 0)),
        pl.BlockSpec((BLK, 1), lambda i: (i, 0)),
        pl.BlockSpec((EMB, H), lambda i: (0, 0)),
        pl.BlockSpec((1, H), lambda i: (0, 0)),
        pl.BlockSpec((H, H), lambda i: (0, 0)),
        pl.BlockSpec((1, H), lambda i: (0, 0)),
        pl.BlockSpec((H, H), lambda i: (0, 0)),
        pl.BlockSpec((1, H), lambda i: (0, 0)),
        pl.BlockSpec((H, 1), lambda i: (0, 0)),
        pl.BlockSpec((1, 1), lambda i: (0, 0)),
    ],
    out_specs=[
        pl.BlockSpec((1, 1), lambda i: (0, 0)),
        pl.BlockSpec((BLK, 1), lambda i: (i, 0)),
    ],
    out_shape=[
        jax.ShapeDtypeStruct((1, 1), jnp.float32),
        jax.ShapeDtypeStruct((B, 1), jnp.float32),
    ],
)


def kernel(input_ids, labels, lengths, table, W1, b1, W2, b2, W3, b3, Wfc, bfc):
    ids_r = ((input_ids & ~(TCOLS - 1))
             + 2 * (input_ids & (THALF - 1))
             + ((input_ids >> LOG_THALF) & 1))
    tlin = _tr_call(table.T)
    pooled = _sc_pool(ids_r, tlin.reshape(VOCAB_PAD, EMB))
    loss2, pred = _mlp_call(
        pooled, lengths, labels.reshape(B, 1),
        W1, b1.reshape(1, H), W2, b2.reshape(1, H),
        W3, b3.reshape(1, H), Wfc, bfc.reshape(1, 1))
    return loss2.reshape(()), pred


# speedup vs baseline: 1.0002x; 1.0002x over previous
import functools

import jax
import jax.numpy as jnp
from jax import lax
from jax.experimental import pallas as pl
from jax.experimental.pallas import tpu as pltpu
from jax.experimental.pallas import tpu_sc as plsc

B = 4096
L = 200
EMB = 64
H = 300

NC, NS, LANES = 2, 16, 16
NW = NC * NS
SEG_PER_W = B // NW
CH0, CH1 = 128, L - 128
ECH = EMB // LANES

_mesh = plsc.VectorSubcoreMesh(core_axis_name="c", subcore_axis_name="s")


@functools.partial(
    pl.kernel,
    out_type=jax.ShapeDtypeStruct((B, EMB), jnp.float32),
    mesh=_mesh,
    scratch_types=[
        pltpu.VMEM((SEG_PER_W, L), jnp.int32),
        pltpu.VMEM((L, EMB), jnp.float32),
        pltpu.VMEM((L, EMB), jnp.float32),
        pltpu.VMEM((L, EMB), jnp.float32),
        pltpu.VMEM((L, EMB), jnp.float32),
        pltpu.VMEM((L, EMB), jnp.float32),
        pltpu.VMEM((L, EMB), jnp.float32),
        pltpu.VMEM((SEG_PER_W, EMB), jnp.float32),
        pltpu.SemaphoreType.DMA,
        pltpu.SemaphoreType.DMA,
        pltpu.SemaphoreType.DMA,
        pltpu.SemaphoreType.DMA,
        pltpu.SemaphoreType.DMA,
        pltpu.SemaphoreType.DMA,
    ],
    compiler_params=pltpu.CompilerParams(use_tc_tiling_on_sc=False),
)
def _sc_pool(ids_hbm, table_hbm, out_hbm, idx_v, buf0, buf1, buf2, buf3,
             buf4, buf5, out_v, sem0, sem1, sem2, sem3, sem4, sem5):
    wid = lax.axis_index("s") * NC + lax.axis_index("c")
    pltpu.sync_copy(ids_hbm.at[pl.ds(wid * SEG_PER_W, SEG_PER_W)], idx_v)

    bufs = (buf0, buf1, buf2, buf3, buf4, buf5)
    sems = (sem0, sem1, sem2, sem3, sem4, sem5)
    NBUF = 6

    def fire(seg, b):
        pltpu.async_copy(table_hbm.at[idx_v.at[seg, pl.ds(0, CH0)]],
                         bufs[b].at[pl.ds(0, CH0)], sems[b])
        pltpu.async_copy(table_hbm.at[idx_v.at[seg, pl.ds(CH0, CH1)]],
                         bufs[b].at[pl.ds(CH0, CH1)], sems[b])

    def drain(b):
        pltpu.make_async_copy(table_hbm.at[pl.ds(0, L)], bufs[b], sems[b]).wait()

    for b in range(NBUF):
        fire(b, b)

    def seg_body(s, b):
        drain(b)
        buf = bufs[b]

        def row8(r8, acc):
            g = [[buf[r8 * 8 + rr, pl.ds(c * LANES, LANES)] for c in range(ECH)]
                 for rr in range(8)]
            while len(g) > 1:
                g = [[a + bb for a, bb in zip(g[2 * k], g[2 * k + 1])]
                     for k in range(len(g) // 2)]
            return tuple(acc[c] + g[0][c] for c in range(ECH))

        acc = tuple(jnp.zeros((LANES,), jnp.float32) for _ in range(ECH))
        acc = lax.fori_loop(0, L // 8, row8, acc)
        for c in range(ECH):
            out_v[s, pl.ds(c * LANES, LANES)] = acc[c]

        nxt = s + NBUF

        @pl.when(nxt < SEG_PER_W)
        def _():
            fire(nxt, b)

    def loop_body(i, carry):
        for b in range(NBUF):
            seg_body(NBUF * i + b, b)
        return carry

    lax.fori_loop(0, SEG_PER_W // NBUF, loop_body, 0)
    for b in range(SEG_PER_W % NBUF):
        seg_body((SEG_PER_W // NBUF) * NBUF + b, b)
    pltpu.sync_copy(out_v, out_hbm.at[pl.ds(wid * SEG_PER_W, SEG_PER_W)])


VOCAB = 1000000
TCOLS = 32768
THALF = TCOLS // 2
LOG_THALF = THALF.bit_length() - 1
TGRID = (VOCAB + TCOLS - 1) // TCOLS
VOCAB_PAD = TGRID * TCOLS


def _tr_body(xt_ref, out_ref):
    x = xt_ref[...]
    xx = jnp.concatenate([x[:, :THALF], x[:, THALF:]], axis=0)
    out_ref[...] = jnp.transpose(xx)


_tr_call = pl.pallas_call(
    _tr_body,
    grid=(TGRID,),
    in_specs=[pl.BlockSpec((EMB, TCOLS), lambda i: (0, i))],
    out_specs=pl.BlockSpec((THALF, 2 * EMB), lambda i: (i, 0)),
    out_shape=jax.ShapeDtypeStruct((TGRID * THALF, 2 * EMB), jnp.float32),
)


BLK = 512
GRID = B // BLK


def _mlp_body(x_ref, len_ref, lab_ref, w1_ref, b1_ref, w2_ref, b2_ref,
              w3_ref, b3_ref, wfc_ref, bfc_ref, loss_ref, pred_ref):
    i = pl.program_id(0)
    bf = jnp.bfloat16
    x = x_ref[...] * (1.0 / len_ref[...])
    h = jnp.tanh(jnp.dot(x.astype(bf), w1_ref[...].astype(bf),
                         preferred_element_type=jnp.float32) + b1_ref[...])
    h = jnp.tanh(jnp.dot(h.astype(bf), w2_ref[...].astype(bf),
                         preferred_element_type=jnp.float32) + b2_ref[...])
    h = jnp.tanh(jnp.dot(h.astype(bf), w3_ref[...].astype(bf),
                         preferred_element_type=jnp.float32) + b3_ref[...])
    z = jnp.dot(h.astype(bf), wfc_ref[...].astype(bf),
                preferred_element_type=jnp.float32) + bfc_ref[...]
    sig = jax.nn.sigmoid(z)
    pred_ref[...] = (sig > 0.5).astype(jnp.float32)
    pv = jnp.clip(sig, 1e-7, 1.0 - 1e-7)
    lv = lab_ref[...]
    part = -jnp.sum(lv * jnp.log(pv) + (1.0 - lv) * jnp.log(1.0 - pv)) * (1.0 / B)

    @pl.when(i == 0)
    def _():
        loss_ref[...] = jnp.zeros((1, 1), jnp.float32)

    loss_ref[...] += jnp.reshape(part, (1, 1))


_mlp_call = pl.pallas_call(
    _mlp_body,
    grid=(GRID,),
    in_specs=[
        pl.BlockSpec((BLK, EMB), lambda i: (i, 0)),
        pl.BlockSpec((BLK, 1), lambda i: (i, 0)),
        pl.BlockSpec((BLK, 1), lambda i: (i, 0)),
        pl.BlockSpec((EMB, H), lambda i: (0, 0)),
        pl.BlockSpec((1, H), lambda i: (0, 0)),
        pl.BlockSpec((H, H), lambda i: (0, 0)),
        pl.BlockSpec((1, H), lambda i: (0, 0)),
        pl.BlockSpec((H, H), lambda i: (0, 0)),
        pl.BlockSpec((1, H), lambda i: (0, 0)),
        pl.BlockSpec((H, 1), lambda i: (0, 0)),
        pl.BlockSpec((1, 1), lambda i: (0, 0)),
    ],
    out_specs=[
        pl.BlockSpec((1, 1), lambda i: (0, 0)),
        pl.BlockSpec((BLK, 1), lambda i: (i, 0)),
    ],
    out_shape=[
        jax.ShapeDtypeStruct((1, 1), jnp.float32),
        jax.ShapeDtypeStruct((B, 1), jnp.float32),
    ],
)


def kernel(input_ids, labels, lengths, table, W1, b1, W2, b2, W3, b3, Wfc, bfc):
    ids_r = ((input_ids & ~(TCOLS - 1))
             + 2 * (input_ids & (THALF - 1))
             + ((input_ids >> LOG_THALF) & 1))
    tlin = _tr_call(table.T)
    pooled = _sc_pool(ids_r, tlin.reshape(VOCAB_PAD, EMB))
    loss2, pred = _mlp_call(
        pooled, lengths, labels.reshape(B, 1),
        W1, b1.reshape(1, H), W2, b2.reshape(1, H),
        W3, b3.reshape(1, H), Wfc, bfc.reshape(1, 1))
    return loss2.reshape(()), pred


# speedup vs baseline: 1.0041x; 1.0039x over previous
import functools

import jax
import jax.numpy as jnp
from jax import lax
from jax.experimental import pallas as pl
from jax.experimental.pallas import tpu as pltpu
from jax.experimental.pallas import tpu_sc as plsc

B = 4096
L = 200
EMB = 64
H = 300

NC, NS, LANES = 2, 16, 16
NW = NC * NS
SEG_PER_W = B // NW
CH0, CH1 = 128, L - 128
ECH = EMB // LANES

_mesh = plsc.VectorSubcoreMesh(core_axis_name="c", subcore_axis_name="s")


@functools.partial(
    pl.kernel,
    out_type=jax.ShapeDtypeStruct((B, EMB), jnp.float32),
    mesh=_mesh,
    scratch_types=[
        pltpu.VMEM((SEG_PER_W, L), jnp.int32),
        pltpu.VMEM((L, EMB), jnp.float32),
        pltpu.VMEM((L, EMB), jnp.float32),
        pltpu.VMEM((L, EMB), jnp.float32),
        pltpu.VMEM((L, EMB), jnp.float32),
        pltpu.VMEM((L, EMB), jnp.float32),
        pltpu.VMEM((L, EMB), jnp.float32),
        pltpu.VMEM((SEG_PER_W, EMB), jnp.float32),
        pltpu.SemaphoreType.DMA,
        pltpu.SemaphoreType.DMA,
        pltpu.SemaphoreType.DMA,
        pltpu.SemaphoreType.DMA,
        pltpu.SemaphoreType.DMA,
        pltpu.SemaphoreType.DMA,
    ],
    compiler_params=pltpu.CompilerParams(use_tc_tiling_on_sc=False),
)
def _sc_pool(ids_hbm, table_hbm, out_hbm, idx_v, buf0, buf1, buf2, buf3,
             buf4, buf5, out_v, sem0, sem1, sem2, sem3, sem4, sem5):
    wid = lax.axis_index("s") * NC + lax.axis_index("c")
    pltpu.sync_copy(ids_hbm.at[pl.ds(wid * SEG_PER_W, SEG_PER_W)], idx_v)

    bufs = (buf0, buf1, buf2, buf3, buf4, buf5)
    sems = (sem0, sem1, sem2, sem3, sem4, sem5)
    NBUF = 6

    def fire(seg, b):
        pltpu.async_copy(table_hbm.at[idx_v.at[seg, pl.ds(0, CH0)]],
                         bufs[b].at[pl.ds(0, CH0)], sems[b])
        pltpu.async_copy(table_hbm.at[idx_v.at[seg, pl.ds(CH0, CH1)]],
                         bufs[b].at[pl.ds(CH0, CH1)], sems[b])

    def drain(b):
        pltpu.make_async_copy(table_hbm.at[pl.ds(0, L)], bufs[b], sems[b]).wait()

    for b in range(NBUF):
        fire(b, b)

    def seg_body(s, b):
        drain(b)
        buf = bufs[b]

        def row8(r8, acc):
            return tuple(tuple(acc[k][c] + buf[r8 * 8 + k, pl.ds(c * LANES, LANES)]
                               for c in range(ECH)) for k in range(8))

        acc = tuple(tuple(jnp.zeros((LANES,), jnp.float32) for _ in range(ECH))
                    for _ in range(8))
        acc = lax.fori_loop(0, L // 8, row8, acc)
        for c in range(ECH):
            f4 = [acc[k][c] + acc[k + 4][c] for k in range(4)]
            f2 = [f4[k] + f4[k + 2] for k in range(2)]
            out_v[s, pl.ds(c * LANES, LANES)] = f2[0] + f2[1]

        nxt = s + NBUF

        @pl.when(nxt < SEG_PER_W)
        def _():
            fire(nxt, b)

    def loop_body(i, carry):
        for b in range(NBUF):
            seg_body(NBUF * i + b, b)
        return carry

    lax.fori_loop(0, SEG_PER_W // NBUF, loop_body, 0)
    for b in range(SEG_PER_W % NBUF):
        seg_body((SEG_PER_W // NBUF) * NBUF + b, b)
    pltpu.sync_copy(out_v, out_hbm.at[pl.ds(wid * SEG_PER_W, SEG_PER_W)])


VOCAB = 1000000
TCOLS = 32768
THALF = TCOLS // 2
LOG_THALF = THALF.bit_length() - 1
TGRID = (VOCAB + TCOLS - 1) // TCOLS
VOCAB_PAD = TGRID * TCOLS


def _tr_body(xt_ref, out_ref):
    x = xt_ref[...]
    xx = jnp.concatenate([x[:, :THALF], x[:, THALF:]], axis=0)
    out_ref[...] = jnp.transpose(xx)


_tr_call = pl.pallas_call(
    _tr_body,
    grid=(TGRID,),
    in_specs=[pl.BlockSpec((EMB, TCOLS), lambda i: (0, i))],
    out_specs=pl.BlockSpec((THALF, 2 * EMB), lambda i: (i, 0)),
    out_shape=jax.ShapeDtypeStruct((TGRID * THALF, 2 * EMB), jnp.float32),
)


BLK = 512
GRID = B // BLK


def _mlp_body(x_ref, len_ref, lab_ref, w1_ref, b1_ref, w2_ref, b2_ref,
              w3_ref, b3_ref, wfc_ref, bfc_ref, loss_ref, pred_ref):
    i = pl.program_id(0)
    bf = jnp.bfloat16
    x = x_ref[...] * (1.0 / len_ref[...])
    h = jnp.tanh(jnp.dot(x.astype(bf), w1_ref[...].astype(bf),
                         preferred_element_type=jnp.float32) + b1_ref[...])
    h = jnp.tanh(jnp.dot(h.astype(bf), w2_ref[...].astype(bf),
                         preferred_element_type=jnp.float32) + b2_ref[...])
    h = jnp.tanh(jnp.dot(h.astype(bf), w3_ref[...].astype(bf),
                         preferred_element_type=jnp.float32) + b3_ref[...])
    z = jnp.dot(h.astype(bf), wfc_ref[...].astype(bf),
                preferred_element_type=jnp.float32) + bfc_ref[...]
    sig = jax.nn.sigmoid(z)
    pred_ref[...] = (sig > 0.5).astype(jnp.float32)
    pv = jnp.clip(sig, 1e-7, 1.0 - 1e-7)
    lv = lab_ref[...]
    part = -jnp.sum(lv * jnp.log(pv) + (1.0 - lv) * jnp.log(1.0 - pv)) * (1.0 / B)

    @pl.when(i == 0)
    def _():
        loss_ref[...] = jnp.zeros((1, 1), jnp.float32)

    loss_ref[...] += jnp.reshape(part, (1, 1))


_mlp_call = pl.pallas_call(
    _mlp_body,
    grid=(GRID,),
    in_specs=[
        pl.BlockSpec((BLK, EMB), lambda i: (i, 0)),
        pl.BlockSpec((BLK, 1), lambda i: (i, 0)),
        pl.BlockSpec((BLK, 1), lambda i: (i, 0)),
        pl.BlockSpec((EMB, H), lambda i: (0, 0)),
        pl.BlockSpec((1, H), lambda i: (0, 0)),
        pl.BlockSpec((H, H), lambda i: (0, 0)),
        pl.BlockSpec((1, H), lambda i: (0, 0)),
        pl.BlockSpec((H, H), lambda i: (0, 0)),
        pl.BlockSpec((1, H), lambda i: (0, 0)),
        pl.BlockSpec((H, 1), lambda i: (0, 0)),
        pl.BlockSpec((1, 1), lambda i: (0, 0)),
    ],
    out_specs=[
        pl.BlockSpec((1, 1), lambda i: (0, 0)),
        pl.BlockSpec((BLK, 1), lambda i: (i, 0)),
    ],
    out_shape=[
        jax.ShapeDtypeStruct((1, 1), jnp.float32),
        jax.ShapeDtypeStruct((B, 1), jnp.float32),
    ],
)


def kernel(input_ids, labels, lengths, table, W1, b1, W2, b2, W3, b3, Wfc, bfc):
    ids_r = ((input_ids & ~(TCOLS - 1))
             + 2 * (input_ids & (THALF - 1))
             + ((input_ids >> LOG_THALF) & 1))
    tlin = _tr_call(table.T)
    pooled = _sc_pool(ids_r, tlin.reshape(VOCAB_PAD, EMB))
    loss2, pred = _mlp_call(
        pooled, lengths, labels.reshape(B, 1),
        W1, b1.reshape(1, H), W2, b2.reshape(1, H),
        W3, b3.reshape(1, H), Wfc, bfc.reshape(1, 1))
    return loss2.reshape(()), pred
